# baseline (speedup 1.0000x reference)
import functools

import jax
import jax.numpy as jnp
import numpy as np
from jax.experimental import pallas as pl
from jax.experimental.pallas import tpu as pltpu
from jax.experimental.pallas import tpu_sc as plsc

_RATIO = 0.5
_RADIUS = 0.2
_K = 64


def _fps_body(n_pts, n_samples, cols, pv_ref, ps_ref, idx_ref, poss_ref):
    px = pv_ref[0]
    py = pv_ref[1]
    pz = pv_ref[2]
    rowi = jax.lax.broadcasted_iota(jnp.int32, (8, cols), 0)
    coli = jax.lax.broadcasted_iota(jnp.int32, (8, cols), 1)
    flat = rowi * cols + coli
    valid = flat < n_pts
    idx_ref[0] = 0
    poss_ref[0, 0] = ps_ref[0, 0]
    poss_ref[1, 0] = ps_ref[1, 0]
    poss_ref[2, 0] = ps_ref[2, 0]
    neg_inf = jnp.float32(-jnp.inf)
    dists0 = jnp.where(valid, jnp.float32(jnp.inf), neg_inf)

    def body(i, carry):
        dists, last = carry
        lx = ps_ref[0, last]
        ly = ps_ref[1, last]
        lz = ps_ref[2, last]
        dx = px - lx
        dy = py - ly
        dz = pz - lz
        d = dx * dx + dy * dy + dz * dz
        dists = jnp.minimum(dists, d)
        m = jnp.max(dists)
        cand = jnp.where(dists == m, flat, jnp.int32(2 ** 30))
        nxt = jnp.min(cand).astype(jnp.int32)
        idx_ref[i] = nxt
        poss_ref[0, i] = ps_ref[0, nxt]
        poss_ref[1, i] = ps_ref[1, nxt]
        poss_ref[2, i] = ps_ref[2, nxt]
        return dists, nxt

    jax.lax.fori_loop(1, n_samples, body, (dists0, jnp.int32(0)))


def _run_fps(pos):
    n_pts = pos.shape[0]
    n_samples = int(n_pts * _RATIO)
    cols = ((n_pts + 8 * 128 - 1) // (8 * 128)) * 128
    pad = 8 * cols - n_pts
    ps = jnp.pad(pos.T, ((0, 0), (0, pad)))
    pv = ps.reshape(3, 8, cols)
    idx, pos_s_t = pl.pallas_call(
        functools.partial(_fps_body, n_pts, n_samples, cols),
        in_specs=[
            pl.BlockSpec(memory_space=pltpu.VMEM),
            pl.BlockSpec(memory_space=pltpu.SMEM),
        ],
        out_specs=[
            pl.BlockSpec(memory_space=pltpu.SMEM),
            pl.BlockSpec(memory_space=pltpu.SMEM),
        ],
        out_shape=[
            jax.ShapeDtypeStruct((n_samples,), jnp.int32),
            jax.ShapeDtypeStruct((3, n_samples), jnp.float32),
        ],
    )(pv, ps)
    return idx, pos_s_t.T


_R2F = np.float32(_RADIUS * _RADIUS)


def _thresh_body(d2_ref, t_ref):
    d2i = jax.lax.bitcast_convert_type(d2_ref[:], jnp.int32)
    r2i = int(np.float32(_R2F).view(np.int32))
    lo0 = jnp.zeros((8, 1), jnp.int32)
    hi0 = jnp.full((8, 1), r2i, jnp.int32)

    def it(_, lohi):
        lo, hi = lohi
        mid = (lo + hi) // 2
        cnt = jnp.sum(jnp.where(d2i <= mid, 1, 0).astype(jnp.int32),
                      axis=1, keepdims=True)
        pred = cnt >= _K
        return jnp.where(pred, lo, mid), jnp.where(pred, mid, hi)

    lo, hi = jax.lax.fori_loop(0, 31, it, (lo0, hi0))
    t_ref[:] = jax.lax.bitcast_convert_type(hi, jnp.float32)


def _run_thresh(d2p):
    n_s, cols = d2p.shape
    return pl.pallas_call(
        _thresh_body,
        grid=(n_s // 8,),
        in_specs=[pl.BlockSpec((8, cols), lambda i: (i, 0))],
        out_specs=pl.BlockSpec((8, 1), lambda i: (i, 0)),
        out_shape=jax.ShapeDtypeStruct((n_s, 1), jnp.float32),
    )(d2p)


_NW = 32
_RPW = 160
_OUTW = 64
_JUNK = 64


def _compact_kernel_fn(n_rows, n_words, slot_hbm, nbr_hbm, srow_v, idx_v):
    n_chunks = n_words // 16
    wid = jax.lax.axis_index("s") * 2 + jax.lax.axis_index("c")
    r0 = wid * _RPW

    def row_body(i, _):
        row = r0 + i

        @pl.when(row < n_rows)
        def _():
            off = pl.multiple_of(row * n_words, 8)
            pltpu.sync_copy(slot_hbm.at[pl.ds(off, n_words)], srow_v)
            for b in range(4):
                idx_v[pl.ds(16 * b, 16)] = jnp.zeros((16,), jnp.int32)

            iv4 = jax.lax.iota(jnp.int32, 16) * 4

            def chunk_body(c, _):
                v32 = srow_v[pl.ds(c * 16, 16)]
                for b in range(4):
                    sb = (v32 >> (8 * b)) & 255
                    plsc.store_scatter(idx_v, [sb], iv4 + (c * 64 + b))
                return 0

            jax.lax.fori_loop(0, n_chunks, chunk_body, 0)
            pltpu.sync_copy(idx_v.at[pl.ds(0, _OUTW)],
                            nbr_hbm.at[pl.ds(row * _OUTW, _OUTW)])

        return 0

    jax.lax.fori_loop(0, _RPW, row_body, 0)


def _run_compact(slots):
    n_rows, n_words = slots.shape
    mesh = plsc.VectorSubcoreMesh(core_axis_name="c", subcore_axis_name="s")
    fn = functools.partial(
        pl.kernel,
        mesh=mesh,
        out_type=jax.ShapeDtypeStruct((_NW * _RPW * _OUTW,), jnp.int32),
        scratch_types=[
            pltpu.VMEM((n_words,), jnp.int32),
            pltpu.VMEM((96,), jnp.int32),
        ],
        compiler_params=pltpu.CompilerParams(needs_layout_passes=False),
    )(functools.partial(_compact_kernel_fn, n_rows, n_words))
    out = fn(slots.reshape(-1)).reshape(_NW * _RPW, _OUTW)
    return out[:n_rows]


def _mm_body(a_ref, b_ref, o_ref):
    o_ref[:] = jnp.dot(a_ref[:], b_ref[:], preferred_element_type=jnp.float32)


def _run_x1(x, W1a):
    n, d = x.shape
    blk = 1024
    npad = ((n + blk - 1) // blk) * blk
    xp = jnp.pad(x, ((0, npad - n), (0, 0)))
    out = pl.pallas_call(
        _mm_body,
        grid=(npad // blk,),
        in_specs=[
            pl.BlockSpec((blk, d), lambda i: (i, 0)),
            pl.BlockSpec((d, 128), lambda i: (0, 0)),
        ],
        out_specs=pl.BlockSpec((blk, 128), lambda i: (i, 0)),
        out_shape=jax.ShapeDtypeStruct((npad, 128), jnp.float32),
    )(xp, W1a)
    return out[:n]


def _mlp_body(c_blk, x1j_ref, rel_ref, msk_ref, w1b_ref, b1_ref, w2_ref,
              b2_ref, w3_ref, b3_ref, o_ref):
    h = x1j_ref[:] + jnp.dot(rel_ref[:], w1b_ref[:],
                             preferred_element_type=jnp.float32) + b1_ref[:]
    h = jnp.maximum(h, 0.0)
    h = jnp.dot(h, w2_ref[:], preferred_element_type=jnp.float32) + b2_ref[:]
    h = jnp.maximum(h, 0.0)
    h = jnp.dot(h, w3_ref[:], preferred_element_type=jnp.float32) + b3_ref[:]
    h = jnp.where(msk_ref[:] > 0, h, jnp.float32(-jnp.inf))
    hm = h.reshape(c_blk, _K, h.shape[-1])
    w = _K
    while w > 1:
        w //= 2
        hm = jnp.maximum(hm[:, :w], hm[:, w:2 * w])
    o_ref[:] = hm[:, 0]


def _run_mlp(x1j, rel8, maskf, W1b8, b1, W2, b2, W3, b3, n_samples):
    c_blk = 8
    rows = c_blk * _K
    d_out = W3.shape[1]
    grid = n_samples // c_blk
    out = pl.pallas_call(
        functools.partial(_mlp_body, c_blk),
        grid=(grid,),
        in_specs=[
            pl.BlockSpec((rows, 128), lambda i: (i, 0)),
            pl.BlockSpec((rows, 8), lambda i: (i, 0)),
            pl.BlockSpec((rows, 1), lambda i: (i, 0)),
            pl.BlockSpec((8, 128), lambda i: (0, 0)),
            pl.BlockSpec((1, 128), lambda i: (0, 0)),
            pl.BlockSpec((128, 128), lambda i: (0, 0)),
            pl.BlockSpec((1, 128), lambda i: (0, 0)),
            pl.BlockSpec((128, d_out), lambda i: (0, 0)),
            pl.BlockSpec((1, d_out), lambda i: (0, 0)),
        ],
        out_specs=pl.BlockSpec((c_blk, d_out), lambda i: (i, 0)),
        out_shape=jax.ShapeDtypeStruct((n_samples, d_out), jnp.float32),
    )(x1j, rel8, maskf, W1b8, b1.reshape(1, -1), W2, b2.reshape(1, -1), W3,
      b3.reshape(1, -1))
    return out


def kernel(x, pos, batch, W1, b1, W2, b2, W3, b3):
    idx, pos_s = _run_fps(pos)

    d2 = (jnp.sum(pos_s ** 2, axis=1)[:, None]
          + jnp.sum(pos ** 2, axis=1)[None, :]
          - 2.0 * (pos_s @ pos.T))
    d2 = jnp.maximum(d2, 0.0)
    d2 = jnp.where(d2 <= _RADIUS * _RADIUS, d2, jnp.inf)

    n_s, n_pts = d2.shape
    cols = ((n_pts + 127) // 128) * 128
    d2p = jnp.pad(d2, ((0, 0), (0, cols - n_pts)),
                  constant_values=jnp.inf)
    thr = _run_thresh(d2p)

    mask = d2p <= thr
    S = jnp.cumsum(mask.astype(jnp.int32), axis=1)
    lane = ((jnp.arange(cols, dtype=jnp.int32) % 64) // 4)[None, :]
    slotv = jnp.where(mask & (S <= _K), S - 1, 80 + lane)
    w = (slotv[:, 0::4] | (slotv[:, 1::4] << 8)
         | (slotv[:, 2::4] << 16) | (slotv[:, 3::4] << 24))
    nbr = _run_compact(w)
    cnt = jnp.minimum(S[:, -1], _K)
    q = jnp.arange(1, _K + 1, dtype=jnp.int32)
    maskq = q[None, :] <= cnt[:, None]
    nbr = jnp.where(maskq, nbr, 0)
    maskf = maskq.astype(jnp.float32).reshape(-1, 1)

    X1 = _run_x1(x, W1[:128])
    flat_nbr = nbr.reshape(-1)
    x1j = X1[flat_nbr]
    rel = pos[flat_nbr] - jnp.broadcast_to(
        pos_s[:, None, :], (pos_s.shape[0], _K, 3)).reshape(-1, 3)
    rel8 = jnp.pad(rel, ((0, 0), (0, 5)))
    W1b8 = jnp.pad(W1[128:131], ((0, 5), (0, 0)))

    out = _run_mlp(x1j, rel8, maskf, W1b8, b1, W2, b2, W3, b3,
                   pos_s.shape[0])
    return out, pos_s, jnp.take(batch, idx)


# speedup vs baseline: 1.7378x; 1.7378x over previous
import functools

import jax
import jax.numpy as jnp
import numpy as np
from jax.experimental import pallas as pl
from jax.experimental.pallas import tpu as pltpu
from jax.experimental.pallas import tpu_sc as plsc

_RATIO = 0.5
_RADIUS = 0.2
_K = 64


def _fps_body(n_pts, n_samples, cols, pv_ref, ps_ref, idx_ref, poss_ref):
    px = pv_ref[0]
    py = pv_ref[1]
    pz = pv_ref[2]
    rowi = jax.lax.broadcasted_iota(jnp.int32, (8, cols), 0)
    coli = jax.lax.broadcasted_iota(jnp.int32, (8, cols), 1)
    flat = rowi * cols + coli
    valid = flat < n_pts
    idx_ref[0] = 0
    poss_ref[0, 0] = ps_ref[0, 0]
    poss_ref[1, 0] = ps_ref[1, 0]
    poss_ref[2, 0] = ps_ref[2, 0]
    neg_inf = jnp.float32(-jnp.inf)
    dists0 = jnp.where(valid, jnp.float32(jnp.inf), neg_inf)

    def body(i, carry):
        dists, last = carry
        lx = ps_ref[0, last]
        ly = ps_ref[1, last]
        lz = ps_ref[2, last]
        dx = px - lx
        dy = py - ly
        dz = pz - lz
        d = dx * dx + dy * dy + dz * dz
        dists = jnp.minimum(dists, d)
        m = jnp.max(dists)
        cand = jnp.where(dists == m, flat, jnp.int32(2 ** 30))
        nxt = jnp.min(cand).astype(jnp.int32)
        idx_ref[i] = nxt
        poss_ref[0, i] = ps_ref[0, nxt]
        poss_ref[1, i] = ps_ref[1, nxt]
        poss_ref[2, i] = ps_ref[2, nxt]
        return dists, nxt

    jax.lax.fori_loop(1, n_samples, body, (dists0, jnp.int32(0)))


def _run_fps(pos):
    n_pts = pos.shape[0]
    n_samples = int(n_pts * _RATIO)
    cols = ((n_pts + 8 * 128 - 1) // (8 * 128)) * 128
    pad = 8 * cols - n_pts
    ps = jnp.pad(pos.T, ((0, 0), (0, pad)))
    pv = ps.reshape(3, 8, cols)
    idx, pos_s_t = pl.pallas_call(
        functools.partial(_fps_body, n_pts, n_samples, cols),
        in_specs=[
            pl.BlockSpec(memory_space=pltpu.VMEM),
            pl.BlockSpec(memory_space=pltpu.SMEM),
        ],
        out_specs=[
            pl.BlockSpec(memory_space=pltpu.SMEM),
            pl.BlockSpec(memory_space=pltpu.SMEM),
        ],
        out_shape=[
            jax.ShapeDtypeStruct((n_samples,), jnp.int32),
            jax.ShapeDtypeStruct((3, n_samples), jnp.float32),
        ],
    )(pv, ps)
    return idx, pos_s_t.T


_R2F = np.float32(_RADIUS * _RADIUS)


def _sel_body(cols, px_ref, py_ref, pz_ref, csx_ref, csy_ref, csz_ref,
              slot_ref, cnt_ref):
    nch = cols // 128
    r2i = int(np.float32(_R2F).view(np.int32))
    big = jnp.float32(jnp.inf)

    cx = csx_ref[:]
    cy = csy_ref[:]
    cz = csz_ref[:]
    dx = px_ref[:] - cx
    dy = py_ref[:] - cy
    dz = pz_ref[:] - cz
    d2w = dx * dx + dy * dy + dz * dz
    d2w = jnp.where(d2w <= _R2F, d2w, big)
    d2i = jax.lax.bitcast_convert_type(d2w, jnp.int32)

    lo0 = jnp.zeros((8, 1), jnp.int32)
    hi0 = jnp.full((8, 1), r2i, jnp.int32)

    def it(_, lohi):
        lo, hi = lohi
        mid = (lo + hi) // 2
        cnt = jnp.sum(jnp.where(d2i <= mid, 1, 0).astype(jnp.int32),
                      axis=1, keepdims=True)
        pred = cnt >= _K
        return jnp.where(pred, lo, mid), jnp.where(pred, mid, hi)

    _, hi = jax.lax.fori_loop(0, 31, it, (lo0, hi0))

    maskw = d2i <= hi
    lane16 = jax.lax.broadcasted_iota(jnp.int32, (8, cols), 1)
    S = maskw.astype(jnp.int32)
    k = 1
    while k < cols:
        S = S + jnp.where(lane16 >= k, pltpu.roll(S, k, 1), 0)
        k *= 2
    lane16 = lane16 % 16
    slot_ref[:] = jnp.where(maskw & (S <= _K), S - 1, 80 + lane16)
    cnt_ref[:] = jnp.minimum(S[:, cols - 1:cols], _K)


def _run_select(pos, n_s, pos_s):
    n_pts = pos.shape[0]
    cols = ((n_pts + 127) // 128) * 128
    nch = cols // 128
    padv = 9.0
    pp = jnp.pad(pos, ((0, cols - n_pts), (0, 0)), constant_values=padv)
    pw = [pp[:, i].reshape(1, cols) for i in range(3)]
    cs = [pos_s[:, i].reshape(n_s, 1) for i in range(3)]

    grid = n_s // 8
    slot, cnt = pl.pallas_call(
        functools.partial(_sel_body, cols),
        grid=(grid,),
        in_specs=[
            pl.BlockSpec((1, cols), lambda i: (0, 0)),
            pl.BlockSpec((1, cols), lambda i: (0, 0)),
            pl.BlockSpec((1, cols), lambda i: (0, 0)),
            pl.BlockSpec((8, 1), lambda i: (i, 0)),
            pl.BlockSpec((8, 1), lambda i: (i, 0)),
            pl.BlockSpec((8, 1), lambda i: (i, 0)),
        ],
        out_specs=[
            pl.BlockSpec((8, cols), lambda i: (i, 0)),
            pl.BlockSpec((8, 1), lambda i: (i, 0)),
        ],
        out_shape=[
            jax.ShapeDtypeStruct((n_s, cols), jnp.int32),
            jax.ShapeDtypeStruct((n_s, 1), jnp.int32),
        ],
    )(pw[0], pw[1], pw[2], cs[0], cs[1], cs[2])
    return slot, cnt.reshape(n_s)


_NW = 32
_RPW = 160
_OUTW = 64
_JUNK = 64


def _compact_kernel_fn(n_rows, n_words, slot_hbm, nbr_hbm, srow_v, idx_v):
    n_chunks = n_words // 16
    wid = jax.lax.axis_index("s") * 2 + jax.lax.axis_index("c")
    r0 = wid * _RPW

    def row_body(i, _):
        row = r0 + i

        @pl.when(row < n_rows)
        def _():
            off = pl.multiple_of(row * n_words, 8)
            pltpu.sync_copy(slot_hbm.at[pl.ds(off, n_words)], srow_v)
            for b in range(4):
                idx_v[pl.ds(16 * b, 16)] = jnp.zeros((16,), jnp.int32)

            iv0 = jax.lax.iota(jnp.int32, 16)

            def chunk_body(c, _):
                slots = srow_v[pl.ds(c * 16, 16)]
                plsc.store_scatter(idx_v, [slots], iv0 + c * 16)
                return 0

            jax.lax.fori_loop(0, n_chunks, chunk_body, 0)
            pltpu.sync_copy(idx_v.at[pl.ds(0, _OUTW)],
                            nbr_hbm.at[pl.ds(row * _OUTW, _OUTW)])

        return 0

    jax.lax.fori_loop(0, _RPW, row_body, 0)


def _run_compact(slots):
    n_rows, n_words = slots.shape
    mesh = plsc.VectorSubcoreMesh(core_axis_name="c", subcore_axis_name="s")
    fn = functools.partial(
        pl.kernel,
        mesh=mesh,
        out_type=jax.ShapeDtypeStruct((_NW * _RPW * _OUTW,), jnp.int32),
        scratch_types=[
            pltpu.VMEM((n_words,), jnp.int32),
            pltpu.VMEM((96,), jnp.int32),
        ],
        compiler_params=pltpu.CompilerParams(needs_layout_passes=False),
    )(functools.partial(_compact_kernel_fn, n_rows, n_words))
    out = fn(slots.reshape(-1)).reshape(_NW * _RPW, _OUTW)
    return out[:n_rows]


def _mm_body(a_ref, b_ref, o_ref):
    o_ref[:] = jnp.dot(a_ref[:], b_ref[:], preferred_element_type=jnp.float32)


def _run_x1(x, W1a):
    n, d = x.shape
    blk = 1024
    npad = ((n + blk - 1) // blk) * blk
    xp = jnp.pad(x, ((0, npad - n), (0, 0)))
    out = pl.pallas_call(
        _mm_body,
        grid=(npad // blk,),
        in_specs=[
            pl.BlockSpec((blk, d), lambda i: (i, 0)),
            pl.BlockSpec((d, 128), lambda i: (0, 0)),
        ],
        out_specs=pl.BlockSpec((blk, 128), lambda i: (i, 0)),
        out_shape=jax.ShapeDtypeStruct((npad, 128), jnp.float32),
    )(xp, W1a)
    return out[:n]


def _mlp_body(c_blk, x1j_ref, rel_ref, msk_ref, w1b_ref, b1_ref, w2_ref,
              b2_ref, w3_ref, b3_ref, o_ref):
    h = x1j_ref[:] + jnp.dot(rel_ref[:], w1b_ref[:],
                             preferred_element_type=jnp.float32) + b1_ref[:]
    h = jnp.maximum(h, 0.0)
    h = jnp.dot(h, w2_ref[:], preferred_element_type=jnp.float32) + b2_ref[:]
    h = jnp.maximum(h, 0.0)
    h = jnp.dot(h, w3_ref[:], preferred_element_type=jnp.float32) + b3_ref[:]
    h = jnp.where(msk_ref[:] > 0, h, jnp.float32(-jnp.inf))
    hm = h.reshape(c_blk, _K, h.shape[-1])
    w = _K
    while w > 1:
        w //= 2
        hm = jnp.maximum(hm[:, :w], hm[:, w:2 * w])
    o_ref[:] = hm[:, 0]


def _run_mlp(x1j, rel8, maskf, W1b8, b1, W2, b2, W3, b3, n_samples):
    c_blk = 8
    rows = c_blk * _K
    d_out = W3.shape[1]
    grid = n_samples // c_blk
    out = pl.pallas_call(
        functools.partial(_mlp_body, c_blk),
        grid=(grid,),
        in_specs=[
            pl.BlockSpec((rows, 128), lambda i: (i, 0)),
            pl.BlockSpec((rows, 8), lambda i: (i, 0)),
            pl.BlockSpec((rows, 1), lambda i: (i, 0)),
            pl.BlockSpec((8, 128), lambda i: (0, 0)),
            pl.BlockSpec((1, 128), lambda i: (0, 0)),
            pl.BlockSpec((128, 128), lambda i: (0, 0)),
            pl.BlockSpec((1, 128), lambda i: (0, 0)),
            pl.BlockSpec((128, d_out), lambda i: (0, 0)),
            pl.BlockSpec((1, d_out), lambda i: (0, 0)),
        ],
        out_specs=pl.BlockSpec((c_blk, d_out), lambda i: (i, 0)),
        out_shape=jax.ShapeDtypeStruct((n_samples, d_out), jnp.float32),
    )(x1j, rel8, maskf, W1b8, b1.reshape(1, -1), W2, b2.reshape(1, -1), W3,
      b3.reshape(1, -1))
    return out


def kernel(x, pos, batch, W1, b1, W2, b2, W3, b3):
    idx, pos_s = _run_fps(pos)

    n_s = pos_s.shape[0]
    slotv, cnt = _run_select(pos, n_s, pos_s)
    nbr = _run_compact(slotv)
    q = jnp.arange(1, _K + 1, dtype=jnp.int32)
    maskq = q[None, :] <= cnt[:, None]
    nbr = jnp.where(maskq, nbr, 0)
    maskf = maskq.astype(jnp.float32).reshape(-1, 1)

    X1 = _run_x1(x, W1[:128])
    flat_nbr = nbr.reshape(-1)
    x1j = X1[flat_nbr]
    rel = pos[flat_nbr] - jnp.broadcast_to(
        pos_s[:, None, :], (pos_s.shape[0], _K, 3)).reshape(-1, 3)
    rel8 = jnp.pad(rel, ((0, 0), (0, 5)))
    W1b8 = jnp.pad(W1[128:131], ((0, 5), (0, 0)))

    out = _run_mlp(x1j, rel8, maskf, W1b8, b1, W2, b2, W3, b3,
                   pos_s.shape[0])
    return out, pos_s, jnp.take(batch, idx)
